# Initial kernel scaffold; baseline (speedup 1.0000x reference)
import jax
import jax.numpy as jnp
from jax.experimental import pallas as pl


def kernel(pos_u, pos_v, neg_v, u_table, v_table):
    raise NotImplementedError("write your pallas kernel here")


# speedup vs baseline: 1.7392x; 1.7392x over previous
import functools

import jax
import jax.numpy as jnp
import numpy as np
from jax import lax
from jax.experimental import pallas as pl
from jax.experimental.pallas import tpu as pltpu
from jax.experimental.pallas import tpu_sc as plsc

B = 16384
V = 1000000
D = 64
NEG = 5

NC = 2
NS = 16
NW = NC * NS
L = 16

BPW = B // NW
C = 128
R = BPW // C
NV = D // L


@functools.partial(
    pl.kernel,
    out_type=[
        jax.ShapeDtypeStruct((B,), jnp.float32),
        jax.ShapeDtypeStruct((NEG * B,), jnp.float32),
    ],
    mesh=plsc.VectorSubcoreMesh(core_axis_name="c", subcore_axis_name="s"),
    compiler_params=pltpu.CompilerParams(
        needs_layout_passes=False, use_tc_tiling_on_sc=False
    ),
    scratch_types=[
        pltpu.VMEM((C,), jnp.int32),
        pltpu.VMEM((C,), jnp.int32),
        pltpu.VMEM((C * NEG,), jnp.int32),
        pltpu.VMEM((C, D), jnp.float32),
        pltpu.VMEM((C, D), jnp.float32),
        pltpu.VMEM((C * NEG, D), jnp.float32),
        pltpu.VMEM((C,), jnp.float32),
        pltpu.VMEM((NEG, C), jnp.float32),
        pltpu.SemaphoreType.DMA,
    ],
)
def _sc_scores(pos_u, pos_v, neg_flat, u_table, v_table, out_pos, out_neg,
               idx_u, idx_v, idx_n, u_rows, v_rows, n_rows, acc_p, acc_n, sem):
    wid = lax.axis_index("s") * NC + lax.axis_index("c")
    iota = lax.iota(jnp.int32, L)

    for r in range(R):
        base = wid * BPW + r * C
        pltpu.sync_copy(pos_u.at[pl.ds(base, C)], idx_u)
        pltpu.sync_copy(pos_v.at[pl.ds(base, C)], idx_v)
        pltpu.sync_copy(neg_flat.at[pl.ds(base * NEG, C * NEG)], idx_n)

        cu = pltpu.async_copy(u_table.at[idx_u], u_rows, sem)
        cv = pltpu.async_copy(v_table.at[idx_v], v_rows, sem)
        cn = pltpu.async_copy(v_table.at[idx_n], n_rows, sem)
        cu.wait()
        cv.wait()
        cn.wait()

        def dot_bcast(u, w):
            p = u[0] * w[0]
            for k in range(1, NV):
                p = p + u[k] * w[k]
            return jnp.full((L,), jnp.sum(p), jnp.float32)

        def load_row(ref, i):
            return [ref[i, pl.ds(L * k, L)] for k in range(NV)]

        def gbody(g, _):
            def jbody(j, accs):
                i = g * L + j
                lane = jnp.equal(iota, j)
                u = load_row(u_rows, i)
                v = load_row(v_rows, i)
                out = [jnp.where(lane, dot_bcast(u, v), accs[0])]
                for n in range(NEG):
                    w = load_row(n_rows, i * NEG + n)
                    out.append(jnp.where(lane, dot_bcast(u, w), accs[1 + n]))
                return tuple(out)

            zero = jnp.zeros((L,), jnp.float32)
            accs = lax.fori_loop(0, L, jbody, (zero,) * (1 + NEG), unroll=4)
            acc_p[pl.ds(g * L, L)] = accs[0]
            for n in range(NEG):
                acc_n[n, pl.ds(g * L, L)] = accs[1 + n]
            return 0

        lax.fori_loop(0, C // L, gbody, 0)

        pltpu.sync_copy(acc_p, out_pos.at[pl.ds(base, C)])
        for n in range(NEG):
            pltpu.sync_copy(acc_n.at[n], out_neg.at[pl.ds(n * B + base, C)])


def _tc_loss_kernel(p_ref, n_ref, o_ref):
    p = jnp.clip(p_ref[...], -10.0, 10.0)
    n = jnp.clip(n_ref[...], -10.0, 10.0)
    loss_pos = jnp.log1p(jnp.exp(-p))
    loss_neg = jnp.log1p(jnp.exp(n))
    o_ref[0, 0] = (jnp.sum(loss_pos) + jnp.sum(loss_neg)) * np.float32(1.0 / B)


def kernel(pos_u, pos_v, neg_v, u_table, v_table):
    pos_u = pos_u.astype(jnp.int32)
    pos_v = pos_v.astype(jnp.int32)
    neg_flat = neg_v.reshape(-1).astype(jnp.int32)

    dots_pos, dots_neg = _sc_scores(pos_u, pos_v, neg_flat, u_table, v_table)

    out = pl.pallas_call(
        _tc_loss_kernel,
        out_shape=jax.ShapeDtypeStruct((1, 1), jnp.float32),
        out_specs=pl.BlockSpec(memory_space=pltpu.SMEM),
    )(dots_pos.reshape(B // 128, 128), dots_neg.reshape(B * NEG // 128, 128))
    return out[0, 0]


# speedup vs baseline: 2.1027x; 1.2090x over previous
import functools

import jax
import jax.numpy as jnp
import numpy as np
from jax import lax
from jax.experimental import pallas as pl
from jax.experimental.pallas import tpu as pltpu
from jax.experimental.pallas import tpu_sc as plsc

B = 16384
V = 1000000
D = 64
NEG = 5

NC = 2
NS = 16
NW = NC * NS
L = 16

BPW = B // NW
C = 16
R = BPW // C
NV = D // L


@functools.partial(
    pl.kernel,
    out_type=[
        jax.ShapeDtypeStruct((B,), jnp.float32),
        jax.ShapeDtypeStruct((NEG * B,), jnp.float32),
    ],
    mesh=plsc.VectorSubcoreMesh(core_axis_name="c", subcore_axis_name="s"),
    compiler_params=pltpu.CompilerParams(needs_layout_passes=False),
    scratch_types=[
        pltpu.VMEM((C,), jnp.int32),
        pltpu.VMEM((C,), jnp.int32),
        pltpu.VMEM((C * NEG,), jnp.int32),
        pltpu.VMEM((8 * C, D), jnp.float32),
        pltpu.VMEM((8 * C, D), jnp.float32),
        pltpu.VMEM((8 * C * NEG, D), jnp.float32),
        pltpu.VMEM((C,), jnp.float32),
        pltpu.VMEM((NEG, C), jnp.float32),
        pltpu.SemaphoreType.DMA,
    ],
)
def _sc_scores(pos_u, pos_v, neg_flat, u_table, v_table, out_pos, out_neg,
               idx_u, idx_v, idx_n, u_rows, v_rows, n_rows,
               acc_p, acc_n, sem):
    wid = lax.axis_index("s") * NC + lax.axis_index("c")
    iota = lax.iota(jnp.int32, L)

    def round_body(r, _):
        base = wid * BPW + r * C
        pltpu.sync_copy(pos_u.at[pl.ds(base, C)], idx_u)
        pltpu.sync_copy(pos_v.at[pl.ds(base, C)], idx_v)
        pltpu.sync_copy(neg_flat.at[pl.ds(base * NEG, C * NEG)], idx_n)

        iu = idx_u[...]
        iv = idx_v[...]
        inn = [idx_n[pl.ds(L * m, L)] for m in range(NEG)]

        def nidx(k):
            return inn[k // L][k % L]

        for i in range(C):
            pu = pl.multiple_of(iu[i] & -8, 8)
            pltpu.async_copy(u_table.at[pl.ds(pu, 8)],
                             u_rows.at[pl.ds(8 * i, 8)], sem)
            pv = pl.multiple_of(iv[i] & -8, 8)
            pltpu.async_copy(v_table.at[pl.ds(pv, 8)],
                             v_rows.at[pl.ds(8 * i, 8)], sem)
            for n in range(NEG):
                k = i * NEG + n
                pn = pl.multiple_of(nidx(k) & -8, 8)
                pltpu.async_copy(v_table.at[pl.ds(pn, 8)],
                                 n_rows.at[pl.ds(8 * k, 8)], sem)

        pltpu.make_async_copy(
            u_table.at[pl.ds(0, 8 * C)], u_rows, sem).wait()
        pltpu.make_async_copy(
            v_table.at[pl.ds(0, 8 * C)], v_rows, sem).wait()
        pltpu.make_async_copy(
            v_table.at[pl.ds(0, 8 * C * NEG)], n_rows, sem).wait()

        def dot_bcast(u, w):
            p = u[0] * w[0]
            for kk in range(1, NV):
                p = p + u[kk] * w[kk]
            return jnp.full((L,), jnp.sum(p), jnp.float32)

        def load_row(ref, row):
            return [ref[row, pl.ds(L * kk, L)] for kk in range(NV)]

        accs = [jnp.zeros((L,), jnp.float32)] * (1 + NEG)
        for j in range(C):
            lane = jnp.equal(iota, j)
            u = load_row(u_rows, 8 * j + (iu[j] & 7))
            v = load_row(v_rows, 8 * j + (iv[j] & 7))
            accs[0] = jnp.where(lane, dot_bcast(u, v), accs[0])
            for n in range(NEG):
                k = j * NEG + n
                w = load_row(n_rows, 8 * k + (nidx(k) & 7))
                accs[1 + n] = jnp.where(lane, dot_bcast(u, w), accs[1 + n])

        acc_p[...] = accs[0]
        for n in range(NEG):
            acc_n[n, :] = accs[1 + n]

        pltpu.sync_copy(acc_p, out_pos.at[pl.ds(base, C)])
        for n in range(NEG):
            pltpu.sync_copy(acc_n.at[n], out_neg.at[pl.ds(n * B + base, C)])
        return 0

    lax.fori_loop(0, R, round_body, 0)


def _tc_loss_kernel(p_ref, n_ref, o_ref):
    p = jnp.clip(p_ref[...], -10.0, 10.0)
    n = jnp.clip(n_ref[...], -10.0, 10.0)
    loss_pos = jnp.log1p(jnp.exp(-p))
    loss_neg = jnp.log1p(jnp.exp(n))
    o_ref[0, 0] = (jnp.sum(loss_pos) + jnp.sum(loss_neg)) * np.float32(1.0 / B)


def kernel(pos_u, pos_v, neg_v, u_table, v_table):
    pos_u = pos_u.astype(jnp.int32)
    pos_v = pos_v.astype(jnp.int32)
    neg_flat = neg_v.reshape(-1).astype(jnp.int32)

    dots_pos, dots_neg = _sc_scores(pos_u, pos_v, neg_flat, u_table, v_table)

    out = pl.pallas_call(
        _tc_loss_kernel,
        out_shape=jax.ShapeDtypeStruct((1, 1), jnp.float32),
        out_specs=pl.BlockSpec(memory_space=pltpu.SMEM),
    )(dots_pos.reshape(B // 128, 128), dots_neg.reshape(B * NEG // 128, 128))
    return out[0, 0]
